# baseline (speedup 1.0000x reference)
import functools

import jax
import jax.numpy as jnp
from jax import lax
from jax.experimental import pallas as pl
from jax.experimental.pallas import tpu as pltpu
from jax.experimental.pallas import tpu_sc as plsc

D = 2048
BATCH = 16384
BM = 512
KC = 512
CH = 32


def _table_kernel(tt_ref, w1_ref, b_ref, t_ref):
    k = pl.program_id(0)
    part = jax.lax.dot_general(
        tt_ref[...], w1_ref[...], (((1,), (1,)), ((), ())),
        preferred_element_type=jnp.float32)

    @pl.when(k == 0)
    def _():
        t_ref[...] = part + b_ref[...]

    @pl.when(k > 0)
    def _():
        t_ref[...] += part


def _main_kernel(emb_ref, add_ref, w2_ref, out_ref):
    acc = jax.lax.dot_general(
        emb_ref[...], w2_ref[...], (((1,), (1,)), ((), ())),
        preferred_element_type=jnp.float32)
    out_ref[...] = acc + add_ref[...]


def kernel(embedding, task_idxs, task_table, W, b):
    n = W.shape[0]
    nt = task_table.shape[0]
    t = pl.pallas_call(
        _table_kernel,
        grid=(D // KC,),
        in_specs=[
            pl.BlockSpec((nt, KC), lambda k: (0, k)),
            pl.BlockSpec((n, KC), lambda k: (0, k)),
            pl.BlockSpec((1, n), lambda k: (0, 0)),
        ],
        out_specs=pl.BlockSpec((nt, n), lambda k: (0, 0)),
        out_shape=jax.ShapeDtypeStruct((nt, n), jnp.float32),
    )(task_table, W, b.reshape(1, n))

    info = plsc.get_sparse_core_info()
    nw = info.num_cores * info.num_subcores
    b_per_w = BATCH // nw
    mesh = plsc.VectorSubcoreMesh(core_axis_name="c", subcore_axis_name="s")

    @functools.partial(
        pl.kernel, mesh=mesh,
        out_type=jax.ShapeDtypeStruct((BATCH, n), jnp.float32),
        scratch_types=[
            pltpu.VMEM((CH,), jnp.int32),
            pltpu.VMEM((CH, n), jnp.float32),
            pltpu.SemaphoreType.DMA,
        ],
    )
    def _sc_gather(t_hbm, idx_hbm, add_hbm, idx_v, rows_v, sem):
        wid = lax.axis_index("s") * info.num_cores + lax.axis_index("c")
        base = wid * b_per_w
        for c in range(b_per_w // CH):
            off = base + c * CH
            pltpu.sync_copy(idx_hbm.at[pl.ds(off, CH)], idx_v)
            pltpu.async_copy(t_hbm.at[idx_v], rows_v, sem).wait()
            pltpu.sync_copy(rows_v, add_hbm.at[pl.ds(off, CH)])

    addend = _sc_gather(t, task_idxs.astype(jnp.int32))

    grid = (BATCH // BM,)
    out = pl.pallas_call(
        _main_kernel,
        grid=grid,
        in_specs=[
            pl.BlockSpec((BM, D), lambda i: (i, 0)),
            pl.BlockSpec((BM, n), lambda i: (i, 0)),
            pl.BlockSpec((n, D), lambda i: (0, 1)),
        ],
        out_specs=pl.BlockSpec((BM, n), lambda i: (i, 0)),
        out_shape=jax.ShapeDtypeStruct((BATCH, n), jnp.float32),
    )(embedding, addend, W)
    return out


# speedup vs baseline: 3.5595x; 3.5595x over previous
import jax
import jax.numpy as jnp
from jax.experimental import pallas as pl

D = 2048
BATCH = 16384
BM = 1024
KC = 512


def _table_kernel(tt_ref, w1_ref, b_ref, t_ref):
    k = pl.program_id(0)
    part = jax.lax.dot_general(
        tt_ref[...], w1_ref[...], (((1,), (1,)), ((), ())),
        preferred_element_type=jnp.float32)

    @pl.when(k == 0)
    def _():
        t_ref[...] = part + b_ref[...]

    @pl.when(k > 0)
    def _():
        t_ref[...] += part


def _main_kernel(emb_ref, idx_ref, t_ref, w2_ref, out_ref):
    acc = jax.lax.dot_general(
        emb_ref[...], w2_ref[...], (((1,), (1,)), ((), ())),
        preferred_element_type=jnp.float32)
    idx = idx_ref[...]
    t = t_ref[...]
    addend = jnp.where(
        idx == 0, t[0:1],
        jnp.where(idx == 1, t[1:2],
                  jnp.where(idx == 2, t[2:3], t[3:4])))
    out_ref[...] = acc + addend


def kernel(embedding, task_idxs, task_table, W, b):
    n = W.shape[0]
    nt = task_table.shape[0]
    t = pl.pallas_call(
        _table_kernel,
        grid=(D // KC,),
        in_specs=[
            pl.BlockSpec((nt, KC), lambda k: (0, k)),
            pl.BlockSpec((n, KC), lambda k: (0, k)),
            pl.BlockSpec((1, n), lambda k: (0, 0)),
        ],
        out_specs=pl.BlockSpec((nt, n), lambda k: (0, 0)),
        out_shape=jax.ShapeDtypeStruct((nt, n), jnp.float32),
    )(task_table, W, b.reshape(1, n))

    idx2d = task_idxs.astype(jnp.int32).reshape(BATCH, 1)

    grid = (BATCH // BM,)
    out = pl.pallas_call(
        _main_kernel,
        grid=grid,
        in_specs=[
            pl.BlockSpec((BM, D), lambda i: (i, 0)),
            pl.BlockSpec((BM, 1), lambda i: (i, 0)),
            pl.BlockSpec((nt, n), lambda i: (0, 0)),
            pl.BlockSpec((n, D), lambda i: (0, 1)),
        ],
        out_specs=pl.BlockSpec((BM, n), lambda i: (i, 0)),
        out_shape=jax.ShapeDtypeStruct((BATCH, n), jnp.float32),
    )(embedding, idx2d, t, W)
    return out
